# baseline (speedup 1.0000x reference)
import functools

import jax
import jax.numpy as jnp
from jax import lax
from jax.experimental import pallas as pl
from jax.experimental.pallas import tpu as pltpu
from jax.experimental.pallas import tpu_sc as plsc

HID = 64
HEADS = 4
C = HID // HEADS
NG = 256

NCORE = 2
NSUB = 16
NW = NCORE * NSUB
EB = 80
AW = 2 * C


def _make_edge_kernel(N, E):
    E2 = 2 * E
    CH = E2 // NW
    mesh = plsc.VectorSubcoreMesh(core_axis_name="c", subcore_axis_name="s")

    @functools.partial(
        pl.kernel,
        out_type=jax.ShapeDtypeStruct((HEADS * NCORE * N, AW), jnp.float32),
        mesh=mesh,
        scratch_types=[
            pltpu.VMEM_SHARED((N, AW), jnp.float32),
            [pltpu.VMEM((EB,), jnp.int32)] * 2,
            [pltpu.VMEM((EB,), jnp.int32)] * 2,
            [pltpu.VMEM((EB,), jnp.int32)] * 2,
            [pltpu.VMEM((EB,), jnp.int32)] * 2,
            [pltpu.VMEM((EB,), jnp.int32)] * 2,
            [pltpu.VMEM((EB,), jnp.int32)] * 2,
            [pltpu.VMEM((EB, C), jnp.float32)] * 2,
            [pltpu.VMEM((EB, C), jnp.float32)] * 2,
            [pltpu.VMEM((EB, C), jnp.float32)] * 2,
            [pltpu.VMEM((EB, C), jnp.float32)] * 2,
            pltpu.VMEM((EB, AW), jnp.float32),
            [pltpu.SemaphoreType.DMA] * 2,
            [pltpu.SemaphoreType.DMA] * 2,
        ],
        compiler_params=pltpu.CompilerParams(
            needs_layout_passes=False, use_tc_tiling_on_sc=False),
    )
    def edge_kernel(q4, k4, v4, e4, srcall, dstall, zrows,
                    out_nd,
                    acc, srcb, dstb, srca, dsta, dsts, eidx,
                    qb, kb, vb, eb, ob, sem_i, sem_g):
        cid = lax.axis_index("c")
        sid = lax.axis_index("s")
        wid = cid * NSUB + sid
        weo = wid * CH
        iota = lax.iota(jnp.int32, 16)
        iota4 = iota * 4
        zvec = jnp.zeros((C,), jnp.float32)
        NB = CH // EB
        ebase = weo - cid * E

        @pl.loop(0, HEADS)
        def _head(h):
            @pl.when(sid == 0)
            def _():
                pltpu.sync_copy(zrows, acc)

            @pl.loop(0, EB)
            def _z(i):
                ob[i, pl.ds(C, C)] = zvec

            plsc.subcore_barrier()

            def issue_idx(bi, d):
                off = weo + bi * EB
                pltpu.async_copy(srcall.at[pl.ds(off, EB)], srcb[d], sem_i[d])
                pltpu.async_copy(dstall.at[pl.ds(off, EB)], dstb[d], sem_i[d])

            def wait_idx(d):
                pltpu.make_async_copy(srcall.at[pl.ds(0, EB)], srcb[d], sem_i[d]).wait()
                pltpu.make_async_copy(dstall.at[pl.ds(0, EB)], dstb[d], sem_i[d]).wait()

            def adj_issue_gathers(bi, d):
                e0 = (ebase + bi * EB) * 4 + h

                @pl.loop(0, EB, step=16)
                def _adj(i):
                    sl = pl.ds(i, 16)
                    srca[d][sl] = srcb[d][sl] * 4 + h
                    dsta[d][sl] = dstb[d][sl] * 4 + h
                    dsts[d][sl] = dstb[d][sl]
                    eidx[d][sl] = (e0 + i * 4) + iota4

                pltpu.async_copy(q4.at[dsta[d]], qb[d], sem_g[d])
                pltpu.async_copy(k4.at[srca[d]], kb[d], sem_g[d])
                pltpu.async_copy(v4.at[srca[d]], vb[d], sem_g[d])
                pltpu.async_copy(e4.at[eidx[d]], eb[d], sem_g[d])

            def wait_gathers(d):
                pltpu.make_async_copy(q4.at[dsta[d]], qb[d], sem_g[d]).wait()
                pltpu.make_async_copy(k4.at[srca[d]], kb[d], sem_g[d]).wait()
                pltpu.make_async_copy(v4.at[srca[d]], vb[d], sem_g[d]).wait()
                pltpu.make_async_copy(e4.at[eidx[d]], eb[d], sem_g[d]).wait()

            def compute_scatter(d):
                @pl.loop(0, EB, step=16)
                def _group(gi):
                    rows = gi + iota
                    alpha = None
                    ecols = []
                    for c in range(C):
                        colc = jnp.full((16,), c, jnp.int32)
                        qc = plsc.load_gather(qb[d], [rows, colc])
                        kc = plsc.load_gather(kb[d], [rows, colc])
                        ec = plsc.load_gather(eb[d], [rows, colc])
                        ecols.append(ec)
                        term = qc * (kc + ec)
                        alpha = term if alpha is None else alpha + term
                    w = jnp.exp(alpha * 0.25)
                    for c in range(C):
                        colc = jnp.full((16,), c, jnp.int32)
                        vc = plsc.load_gather(vb[d], [rows, colc])
                        plsc.store_scatter(ob, [rows, colc], w * (vc + ecols[c]))
                    hcol = jnp.full((16,), C, jnp.int32) + h
                    plsc.store_scatter(ob, [rows, hcol], w)

                pltpu.sync_copy(ob, acc.at[dsts[d]], add=True)

            issue_idx(0, 0)
            issue_idx(1, 1)
            wait_idx(0)
            adj_issue_gathers(0, 0)
            issue_idx(2, 0)

            @pl.loop(0, NB - 1, step=2)
            def _block(b):
                wait_idx(1)
                adj_issue_gathers(b + 1, 1)

                @pl.when(b < NB - 3)
                def _():
                    issue_idx(b + 3, 1)

                wait_gathers(0)
                compute_scatter(0)
                wait_idx(0)
                adj_issue_gathers(b + 2, 0)

                @pl.when(b < NB - 3)
                def _():
                    issue_idx(b + 4, 0)

                wait_gathers(1)
                compute_scatter(1)

            wait_gathers(0)
            compute_scatter(0)

            plsc.subcore_barrier()

            @pl.when(sid == 0)
            def _():
                pltpu.sync_copy(acc, out_nd.at[pl.ds((h * NCORE + cid) * N, N)])

            plsc.subcore_barrier()

    return edge_kernel


def kernel(x, edge_index, edge_attr, batch, params):
    p = params
    N = x.shape[0]
    E = edge_attr.shape[0]

    nf = jax.nn.relu(jax.nn.relu(x @ p['nW0'] + p['nb0']) @ p['nW1'] + p['nb1'])
    ef = jax.nn.relu(jax.nn.relu(edge_attr @ p['eW0'] + p['eb0']) @ p['eW1'] + p['eb1'])
    e = ef @ p['We']

    q = nf @ p['Wq'] + p['bq']
    k = nf @ p['Wk'] + p['bk']
    v = nf @ p['Wv'] + p['bv']

    q4 = q.reshape(HEADS * N, C)
    k4 = k.reshape(HEADS * N, C)
    v4 = v.reshape(HEADS * N, C)
    e4 = e.reshape(HEADS * E, C)

    srcall = jnp.concatenate([edge_index[0], edge_index[1]], axis=0)
    dstall = jnp.concatenate([edge_index[1], edge_index[0]], axis=0)
    zrows = jnp.zeros((N, AW), jnp.float32)

    out_nd = _make_edge_kernel(N, E)(
        q4, k4, v4, e4, srcall, dstall, zrows)

    nd = out_nd.reshape(HEADS, NCORE, N, AW).sum(axis=1)
    numer = nd[..., :C]
    denom = jnp.stack([nd[h, :, C + h] for h in range(HEADS)])[..., None]
    out = (numer / (denom + 1e-16)).transpose(1, 0, 2).reshape(N, HID)

    out = out + nf @ p['Wskip'] + p['bskip']
    mu = out.mean(axis=-1, keepdims=True)
    var = ((out - mu) ** 2).mean(axis=-1, keepdims=True)
    nfn = (out - mu) / jnp.sqrt(var + 1e-5) * p['ln_g'] + p['ln_b']
    nf2 = nfn + nf

    ones = jnp.ones((N,), jnp.float32)
    cnt = jax.ops.segment_sum(ones, batch, num_segments=NG)
    gf = jax.ops.segment_sum(nf2, batch, num_segments=NG) / jnp.clip(cnt, 1.0)[:, None]
    cls = (jax.nn.relu(gf @ p['cW0'] + p['cb0']) @ p['cW1'] + p['cb1']).squeeze(-1)
    reg = (jax.nn.relu(gf @ p['rW0'] + p['rb0']) @ p['rW1'] + p['rb1']).squeeze(-1)
    return (cls, reg)
